# baseline (speedup 1.0000x reference)
import jax
import jax.numpy as jnp
from jax.experimental import pallas as pl
from jax.experimental.pallas import tpu as pltpu

_T = 256


def _xw_kernel(x_ref, w_ref, o_ref):
    x = x_ref[...].astype(jnp.bfloat16)
    o_ref[...] = jnp.dot(
        x, w_ref[...], preferred_element_type=jnp.float32
    ).astype(jnp.bfloat16)


def _h_kernel(a_ref, p_ref, w2_ref, b1_ref, d2_ref, q_ref):
    i = pl.program_id(0)
    h = jnp.dot(a_ref[...], p_ref[...], preferred_element_type=jnp.float32)
    p_i = p_ref[pl.ds(i * _T, _T), :].astype(jnp.float32)
    d2_i = d2_ref[pl.ds(i * _T, _T), :]
    h = jnp.maximum(h + d2_i * p_i + b1_ref[...], 0.0)
    q_ref[...] = jnp.dot(
        h.astype(jnp.bfloat16), w2_ref[...], preferred_element_type=jnp.float32
    ).astype(jnp.bfloat16)


def _out_kernel(a_ref, q_ref, b2_ref, d2_ref, o_ref):
    i = pl.program_id(0)
    o = jnp.dot(a_ref[...], q_ref[...], preferred_element_type=jnp.float32)
    q_i = q_ref[pl.ds(i * _T, _T), :].astype(jnp.float32)
    d2_i = d2_ref[pl.ds(i * _T, _T), :]
    o_ref[...] = o + d2_i * q_i + b2_ref[...]


def kernel(x, edge_index, w1, b1, w2, b2):
    n, f_in = x.shape
    hidden = w1.shape[1]
    c = w2.shape[1]
    c_pad = max(128, ((c + 127) // 128) * 128)

    src = edge_index[0]
    dst = edge_index[1]
    cnt = jnp.zeros((n,), jnp.float32).at[dst].add(1.0)
    dis = 1.0 / jnp.sqrt(cnt + 1.0)
    d2 = (dis * dis).reshape(n, 1)
    vals = (dis[dst] * dis[src]).astype(jnp.bfloat16)
    s = jnp.zeros((n, n), jnp.bfloat16).at[dst, src].add(vals)

    w1b = w1.astype(jnp.bfloat16)
    w2p = jnp.zeros((hidden, c_pad), jnp.bfloat16).at[:, :c].set(
        w2.astype(jnp.bfloat16))
    b2p = jnp.zeros((1, c_pad), jnp.float32).at[:, :c].set(b2)

    grid = (n // _T,)
    par = pltpu.CompilerParams(dimension_semantics=("parallel",))

    p = pl.pallas_call(
        _xw_kernel,
        out_shape=jax.ShapeDtypeStruct((n, hidden), jnp.bfloat16),
        grid=grid,
        in_specs=[
            pl.BlockSpec((_T, f_in), lambda i: (i, 0)),
            pl.BlockSpec((f_in, hidden), lambda i: (0, 0)),
        ],
        out_specs=pl.BlockSpec((_T, hidden), lambda i: (i, 0)),
        compiler_params=par,
    )(x, w1b)

    q = pl.pallas_call(
        _h_kernel,
        out_shape=jax.ShapeDtypeStruct((n, c_pad), jnp.bfloat16),
        grid=grid,
        in_specs=[
            pl.BlockSpec((_T, n), lambda i: (i, 0)),
            pl.BlockSpec((n, hidden), lambda i: (0, 0)),
            pl.BlockSpec((hidden, c_pad), lambda i: (0, 0)),
            pl.BlockSpec((1, hidden), lambda i: (0, 0)),
            pl.BlockSpec((n, 1), lambda i: (0, 0)),
        ],
        out_specs=pl.BlockSpec((_T, c_pad), lambda i: (i, 0)),
        compiler_params=par,
    )(s, p, w2p, b1, d2)

    out = pl.pallas_call(
        _out_kernel,
        out_shape=jax.ShapeDtypeStruct((n, c_pad), jnp.float32),
        grid=grid,
        in_specs=[
            pl.BlockSpec((_T, n), lambda i: (i, 0)),
            pl.BlockSpec((n, c_pad), lambda i: (0, 0)),
            pl.BlockSpec((1, c_pad), lambda i: (0, 0)),
            pl.BlockSpec((n, 1), lambda i: (0, 0)),
        ],
        out_specs=pl.BlockSpec((_T, c_pad), lambda i: (i, 0)),
        compiler_params=par,
    )(s, q, b2p, d2)

    return out[:, :c]


# speedup vs baseline: 2.1258x; 2.1258x over previous
import jax
import jax.numpy as jnp
from jax.experimental import pallas as pl
from jax.experimental.pallas import tpu as pltpu

_T = 256


def _xw_kernel(x_ref, w_ref, o_ref):
    x = x_ref[...].astype(jnp.bfloat16)
    o_ref[...] = jnp.dot(
        x, w_ref[...], preferred_element_type=jnp.float32
    ).astype(jnp.bfloat16)


def _h_kernel(a_ref, p_ref, w2_ref, b1_ref, dis_ref, q_ref):
    i = pl.program_id(0)
    h = jnp.dot(a_ref[...], p_ref[...], preferred_element_type=jnp.float32)
    p_i = p_ref[pl.ds(i * _T, _T), :].astype(jnp.float32)
    dis_i = dis_ref[pl.ds(i * _T, _T), :]
    h = jnp.maximum(dis_i * (h + p_i) + b1_ref[...], 0.0)
    q = jnp.dot(h.astype(jnp.bfloat16), w2_ref[...],
                preferred_element_type=jnp.float32)
    q_ref[...] = (dis_i * q).astype(jnp.bfloat16)


def _out_kernel(a_ref, q_ref, b2_ref, dis_ref, o_ref):
    i = pl.program_id(0)
    o = jnp.dot(a_ref[...], q_ref[...], preferred_element_type=jnp.float32)
    q_i = q_ref[pl.ds(i * _T, _T), :].astype(jnp.float32)
    dis_i = dis_ref[pl.ds(i * _T, _T), :]
    o_ref[...] = dis_i * (o + q_i) + b2_ref[...]


def kernel(x, edge_index, w1, b1, w2, b2):
    n, f_in = x.shape
    hidden = w1.shape[1]
    c = w2.shape[1]
    c_pad = max(128, ((c + 127) // 128) * 128)

    src = edge_index[0]
    dst = edge_index[1]
    a = jnp.zeros((n, n), jnp.bfloat16).at[dst, src].add(
        jnp.ones((src.shape[0],), jnp.bfloat16))
    deg = jnp.sum(a, axis=1, dtype=jnp.float32) + 1.0
    dis = (1.0 / jnp.sqrt(deg)).reshape(n, 1)

    w1b = w1.astype(jnp.bfloat16)
    w2p = jnp.zeros((hidden, c_pad), jnp.bfloat16).at[:, :c].set(
        w2.astype(jnp.bfloat16))
    b2p = jnp.zeros((1, c_pad), jnp.float32).at[:, :c].set(b2)

    grid = (n // _T,)
    par = pltpu.CompilerParams(dimension_semantics=("parallel",))

    p = pl.pallas_call(
        _xw_kernel,
        out_shape=jax.ShapeDtypeStruct((n, hidden), jnp.bfloat16),
        grid=grid,
        in_specs=[
            pl.BlockSpec((_T, f_in), lambda i: (i, 0)),
            pl.BlockSpec((f_in, hidden), lambda i: (0, 0)),
        ],
        out_specs=pl.BlockSpec((_T, hidden), lambda i: (i, 0)),
        compiler_params=par,
    )(x, w1b)

    p_s = (p.astype(jnp.float32) * dis).astype(jnp.bfloat16)

    q = pl.pallas_call(
        _h_kernel,
        out_shape=jax.ShapeDtypeStruct((n, c_pad), jnp.bfloat16),
        grid=grid,
        in_specs=[
            pl.BlockSpec((_T, n), lambda i: (i, 0)),
            pl.BlockSpec((n, hidden), lambda i: (0, 0)),
            pl.BlockSpec((hidden, c_pad), lambda i: (0, 0)),
            pl.BlockSpec((1, hidden), lambda i: (0, 0)),
            pl.BlockSpec((n, 1), lambda i: (0, 0)),
        ],
        out_specs=pl.BlockSpec((_T, c_pad), lambda i: (i, 0)),
        compiler_params=par,
    )(a, p_s, w2p, b1, dis)

    out = pl.pallas_call(
        _out_kernel,
        out_shape=jax.ShapeDtypeStruct((n, c_pad), jnp.float32),
        grid=grid,
        in_specs=[
            pl.BlockSpec((_T, n), lambda i: (i, 0)),
            pl.BlockSpec((n, c_pad), lambda i: (0, 0)),
            pl.BlockSpec((1, c_pad), lambda i: (0, 0)),
            pl.BlockSpec((n, 1), lambda i: (0, 0)),
        ],
        out_specs=pl.BlockSpec((_T, c_pad), lambda i: (i, 0)),
        compiler_params=par,
    )(a, q, b2p, dis)

    return out[:, :c]


# speedup vs baseline: 3.4675x; 1.6311x over previous
import jax
import jax.numpy as jnp
from jax.experimental import pallas as pl
from jax.experimental.pallas import tpu as pltpu

_T = 256


def _xw_kernel(x_ref, w_ref, o_ref):
    x = x_ref[...].astype(jnp.bfloat16)
    o_ref[...] = jnp.dot(
        x, w_ref[...], preferred_element_type=jnp.float32
    ).astype(jnp.bfloat16)


def _h_kernel(a_ref, p_ref, w2_ref, b1_ref, dis_ref, q_ref):
    i = pl.program_id(0)
    h = jnp.dot(a_ref[...], p_ref[...], preferred_element_type=jnp.float32)
    p_i = p_ref[pl.ds(i * _T, _T), :].astype(jnp.float32)
    dis_i = dis_ref[pl.ds(i * _T, _T), :]
    h = jnp.maximum(dis_i * (h + p_i) + b1_ref[...], 0.0)
    q = jnp.dot(h.astype(jnp.bfloat16), w2_ref[...],
                preferred_element_type=jnp.float32)
    q_ref[...] = (dis_i * q).astype(jnp.bfloat16)


def _out_kernel(a_ref, q_ref, b2_ref, dis_ref, o_ref):
    i = pl.program_id(0)
    o = jnp.dot(a_ref[...], q_ref[...], preferred_element_type=jnp.float32)
    q_i = q_ref[pl.ds(i * _T, _T), :].astype(jnp.float32)
    dis_i = dis_ref[pl.ds(i * _T, _T), :]
    o_ref[...] = dis_i * (o + q_i) + b2_ref[...]


def kernel(x, edge_index, w1, b1, w2, b2):
    n, f_in = x.shape
    hidden = w1.shape[1]
    c = w2.shape[1]
    c_pad = max(128, ((c + 127) // 128) * 128)

    src = edge_index[0]
    dst = edge_index[1]
    a32 = jnp.zeros((n, n), jnp.float32).at[dst, src].add(1.0)
    a = a32.astype(jnp.bfloat16)
    deg = jnp.sum(a32, axis=1) + 1.0
    dis = (1.0 / jnp.sqrt(deg)).reshape(n, 1)

    w1b = w1.astype(jnp.bfloat16)
    w2p = jnp.zeros((hidden, c_pad), jnp.bfloat16).at[:, :c].set(
        w2.astype(jnp.bfloat16))
    b2p = jnp.zeros((1, c_pad), jnp.float32).at[:, :c].set(b2)

    grid = (n // _T,)
    par = pltpu.CompilerParams(dimension_semantics=("parallel",))

    p = pl.pallas_call(
        _xw_kernel,
        out_shape=jax.ShapeDtypeStruct((n, hidden), jnp.bfloat16),
        grid=grid,
        in_specs=[
            pl.BlockSpec((_T, f_in), lambda i: (i, 0)),
            pl.BlockSpec((f_in, hidden), lambda i: (0, 0)),
        ],
        out_specs=pl.BlockSpec((_T, hidden), lambda i: (i, 0)),
        compiler_params=par,
    )(x, w1b)

    p_s = (p.astype(jnp.float32) * dis).astype(jnp.bfloat16)

    q = pl.pallas_call(
        _h_kernel,
        out_shape=jax.ShapeDtypeStruct((n, c_pad), jnp.bfloat16),
        grid=grid,
        in_specs=[
            pl.BlockSpec((_T, n), lambda i: (i, 0)),
            pl.BlockSpec((n, hidden), lambda i: (0, 0)),
            pl.BlockSpec((hidden, c_pad), lambda i: (0, 0)),
            pl.BlockSpec((1, hidden), lambda i: (0, 0)),
            pl.BlockSpec((n, 1), lambda i: (0, 0)),
        ],
        out_specs=pl.BlockSpec((_T, c_pad), lambda i: (i, 0)),
        compiler_params=par,
    )(a, p_s, w2p, b1, dis)

    out = pl.pallas_call(
        _out_kernel,
        out_shape=jax.ShapeDtypeStruct((n, c_pad), jnp.float32),
        grid=grid,
        in_specs=[
            pl.BlockSpec((_T, n), lambda i: (i, 0)),
            pl.BlockSpec((n, c_pad), lambda i: (0, 0)),
            pl.BlockSpec((1, c_pad), lambda i: (0, 0)),
            pl.BlockSpec((n, 1), lambda i: (0, 0)),
        ],
        out_specs=pl.BlockSpec((_T, c_pad), lambda i: (i, 0)),
        compiler_params=par,
    )(a, q, b2p, dis)

    return out[:, :c]


# speedup vs baseline: 3.5030x; 1.0102x over previous
import jax
import jax.numpy as jnp
from jax.experimental import pallas as pl
from jax.experimental.pallas import tpu as pltpu

_T = 256


def _xw_kernel(x_ref, w_ref, o_ref):
    x = x_ref[...].astype(jnp.bfloat16)
    o_ref[...] = jnp.dot(
        x, w_ref[...], preferred_element_type=jnp.float32
    ).astype(jnp.bfloat16)


def _h_kernel(a_ref, p_ref, w2_ref, b1_ref, dis_ref, q_ref):
    i = pl.program_id(0)
    h = jnp.dot(a_ref[...], p_ref[...], preferred_element_type=jnp.float32)
    p_i = p_ref[pl.ds(i * _T, _T), :].astype(jnp.float32)
    dis_i = dis_ref[pl.ds(i * _T, _T), :]
    h = jnp.maximum(dis_i * (h + p_i) + b1_ref[...], 0.0)
    q = jnp.dot(h.astype(jnp.bfloat16), w2_ref[...],
                preferred_element_type=jnp.float32)
    q_ref[...] = (dis_i * q).astype(jnp.bfloat16)


def _out_kernel(a_ref, q_ref, b2_ref, dis_ref, o_ref):
    i = pl.program_id(0)
    o = jnp.dot(a_ref[...], q_ref[...], preferred_element_type=jnp.float32)
    q_i = q_ref[pl.ds(i * _T, _T), :].astype(jnp.float32)
    dis_i = dis_ref[pl.ds(i * _T, _T), :]
    o_ref[...] = dis_i * (o + q_i) + b2_ref[...]


def kernel(x, edge_index, w1, b1, w2, b2):
    n, f_in = x.shape
    hidden = w1.shape[1]
    c = w2.shape[1]
    c_pad = max(128, ((c + 127) // 128) * 128)

    src = edge_index[0]
    dst = edge_index[1]
    lin = dst * n + src
    a32 = jnp.zeros((n * n,), jnp.float32).at[lin].add(1.0).reshape(n, n)
    a = a32.astype(jnp.bfloat16)
    deg = jnp.sum(a32, axis=1) + 1.0
    dis = (1.0 / jnp.sqrt(deg)).reshape(n, 1)

    w1b = w1.astype(jnp.bfloat16)
    w2p = jnp.zeros((hidden, c_pad), jnp.bfloat16).at[:, :c].set(
        w2.astype(jnp.bfloat16))
    b2p = jnp.zeros((1, c_pad), jnp.float32).at[:, :c].set(b2)

    grid = (n // _T,)
    par = pltpu.CompilerParams(dimension_semantics=("parallel",))

    p = pl.pallas_call(
        _xw_kernel,
        out_shape=jax.ShapeDtypeStruct((n, hidden), jnp.bfloat16),
        grid=grid,
        in_specs=[
            pl.BlockSpec((_T, f_in), lambda i: (i, 0)),
            pl.BlockSpec((f_in, hidden), lambda i: (0, 0)),
        ],
        out_specs=pl.BlockSpec((_T, hidden), lambda i: (i, 0)),
        compiler_params=par,
    )(x, w1b)

    p_s = (p.astype(jnp.float32) * dis).astype(jnp.bfloat16)

    q = pl.pallas_call(
        _h_kernel,
        out_shape=jax.ShapeDtypeStruct((n, c_pad), jnp.bfloat16),
        grid=grid,
        in_specs=[
            pl.BlockSpec((_T, n), lambda i: (i, 0)),
            pl.BlockSpec((n, hidden), lambda i: (0, 0)),
            pl.BlockSpec((hidden, c_pad), lambda i: (0, 0)),
            pl.BlockSpec((1, hidden), lambda i: (0, 0)),
            pl.BlockSpec((n, 1), lambda i: (0, 0)),
        ],
        out_specs=pl.BlockSpec((_T, c_pad), lambda i: (i, 0)),
        compiler_params=par,
    )(a, p_s, w2p, b1, dis)

    out = pl.pallas_call(
        _out_kernel,
        out_shape=jax.ShapeDtypeStruct((n, c_pad), jnp.float32),
        grid=grid,
        in_specs=[
            pl.BlockSpec((_T, n), lambda i: (i, 0)),
            pl.BlockSpec((n, c_pad), lambda i: (0, 0)),
            pl.BlockSpec((1, c_pad), lambda i: (0, 0)),
            pl.BlockSpec((n, 1), lambda i: (0, 0)),
        ],
        out_specs=pl.BlockSpec((_T, c_pad), lambda i: (i, 0)),
        compiler_params=par,
    )(a, q, b2p, dis)

    return out[:, :c]


# speedup vs baseline: 5.0628x; 1.4453x over previous
import jax
import jax.numpy as jnp
from jax.experimental import pallas as pl
from jax.experimental.pallas import tpu as pltpu

_T = 256
_CH = 512


def _build_a_kernel(bounds_ref, rl_ref, cl_ref, at_ref, deg_ref, acc_ref):
    i = pl.program_id(0)
    n = at_ref.shape[0]
    acc_ref[...] = jnp.zeros_like(acc_ref)
    k0 = bounds_ref[i] // _CH
    k1 = (bounds_ref[i + 1] + _CH - 1) // _CH

    row_iota = jax.lax.broadcasted_iota(jnp.int32, (_T, _CH), 0)
    col_iota = jax.lax.broadcasted_iota(jnp.int32, (n, _CH), 0)

    def body(k, _):
        rv = rl_ref[pl.ds(k, 1), :] - i * _T
        cv = cl_ref[pl.ds(k, 1), :]
        d_oh = (row_iota == rv).astype(jnp.bfloat16)
        s_oh = (col_iota == cv).astype(jnp.bfloat16)
        acc_ref[...] += jax.lax.dot_general(
            s_oh, d_oh, (((1,), (1,)), ((), ())),
            preferred_element_type=jnp.float32)
        return _

    jax.lax.fori_loop(k0, k1, body, 0)
    acc = acc_ref[...]
    at_ref[...] = acc.astype(jnp.bfloat16)
    deg_ref[...] = jnp.sum(acc, axis=0, keepdims=True)


def _xw_kernel(x_ref, w_ref, o_ref):
    x = x_ref[...].astype(jnp.bfloat16)
    o_ref[...] = jnp.dot(
        x, w_ref[...], preferred_element_type=jnp.float32
    ).astype(jnp.bfloat16)


def _h_kernel(at_ref, p_ref, w2_ref, b1_ref, dis_ref, q_ref):
    i = pl.program_id(0)
    h = jax.lax.dot_general(
        at_ref[...], p_ref[...], (((0,), (0,)), ((), ())),
        preferred_element_type=jnp.float32)
    p_i = p_ref[pl.ds(i * _T, _T), :].astype(jnp.float32)
    dis_i = dis_ref[pl.ds(i * _T, _T), :]
    h = jnp.maximum(dis_i * (h + p_i) + b1_ref[...], 0.0)
    q = jnp.dot(h.astype(jnp.bfloat16), w2_ref[...],
                preferred_element_type=jnp.float32)
    q_ref[...] = (dis_i * q).astype(jnp.bfloat16)


def _out_kernel(at_ref, q_ref, b2_ref, dis_ref, o_ref):
    i = pl.program_id(0)
    o = jax.lax.dot_general(
        at_ref[...], q_ref[...], (((0,), (0,)), ((), ())),
        preferred_element_type=jnp.float32)
    q_i = q_ref[pl.ds(i * _T, _T), :].astype(jnp.float32)
    dis_i = dis_ref[pl.ds(i * _T, _T), :]
    o_ref[...] = dis_i * (o + q_i) + b2_ref[...]


def kernel(x, edge_index, w1, b1, w2, b2):
    n, f_in = x.shape
    hidden = w1.shape[1]
    c = w2.shape[1]
    c_pad = max(128, ((c + 127) // 128) * 128)
    e = edge_index.shape[1]
    n_tiles = n // _T
    e_pad = ((e + _CH - 1) // _CH) * _CH

    src = edge_index[0]
    dst = edge_index[1]
    lin = jnp.sort(jnp.concatenate(
        [dst * n + src, jnp.full((e_pad - e,), n * n, jnp.int32)]))
    bounds = jnp.searchsorted(
        lin, jnp.arange(n_tiles + 1, dtype=jnp.int32) * (_T * n)
    ).astype(jnp.int32)
    rl = (lin // n).reshape(e_pad // _CH, _CH)
    cl = (lin % n).reshape(e_pad // _CH, _CH)

    w1b = w1.astype(jnp.bfloat16)
    w2p = jnp.zeros((hidden, c_pad), jnp.bfloat16).at[:, :c].set(
        w2.astype(jnp.bfloat16))
    b2p = jnp.zeros((1, c_pad), jnp.float32).at[:, :c].set(b2)

    grid = (n_tiles,)
    par = pltpu.CompilerParams(dimension_semantics=("parallel",))

    a_t, deg = pl.pallas_call(
        _build_a_kernel,
        out_shape=(jax.ShapeDtypeStruct((n, n), jnp.bfloat16),
                   jax.ShapeDtypeStruct((1, n), jnp.float32)),
        grid=grid,
        in_specs=[
            pl.BlockSpec(memory_space=pltpu.SMEM),
            pl.BlockSpec((e_pad // _CH, _CH), lambda i: (0, 0)),
            pl.BlockSpec((e_pad // _CH, _CH), lambda i: (0, 0)),
        ],
        out_specs=(pl.BlockSpec((n, _T), lambda i: (0, i)),
                   pl.BlockSpec((1, _T), lambda i: (0, i))),
        scratch_shapes=[pltpu.VMEM((n, _T), jnp.float32)],
        compiler_params=par,
    )(bounds, rl, cl)

    dis = (1.0 / jnp.sqrt(deg + 1.0)).reshape(n, 1)

    p = pl.pallas_call(
        _xw_kernel,
        out_shape=jax.ShapeDtypeStruct((n, hidden), jnp.bfloat16),
        grid=grid,
        in_specs=[
            pl.BlockSpec((_T, f_in), lambda i: (i, 0)),
            pl.BlockSpec((f_in, hidden), lambda i: (0, 0)),
        ],
        out_specs=pl.BlockSpec((_T, hidden), lambda i: (i, 0)),
        compiler_params=par,
    )(x, w1b)

    p_s = (p.astype(jnp.float32) * dis).astype(jnp.bfloat16)

    q = pl.pallas_call(
        _h_kernel,
        out_shape=jax.ShapeDtypeStruct((n, c_pad), jnp.bfloat16),
        grid=grid,
        in_specs=[
            pl.BlockSpec((n, _T), lambda i: (0, i)),
            pl.BlockSpec((n, hidden), lambda i: (0, 0)),
            pl.BlockSpec((hidden, c_pad), lambda i: (0, 0)),
            pl.BlockSpec((1, hidden), lambda i: (0, 0)),
            pl.BlockSpec((n, 1), lambda i: (0, 0)),
        ],
        out_specs=pl.BlockSpec((_T, c_pad), lambda i: (i, 0)),
        compiler_params=par,
    )(a_t, p_s, w2p, b1, dis)

    out = pl.pallas_call(
        _out_kernel,
        out_shape=jax.ShapeDtypeStruct((n, c_pad), jnp.float32),
        grid=grid,
        in_specs=[
            pl.BlockSpec((n, _T), lambda i: (0, i)),
            pl.BlockSpec((n, c_pad), lambda i: (0, 0)),
            pl.BlockSpec((1, c_pad), lambda i: (0, 0)),
            pl.BlockSpec((n, 1), lambda i: (0, 0)),
        ],
        out_specs=pl.BlockSpec((_T, c_pad), lambda i: (i, 0)),
        compiler_params=par,
    )(a_t, q, b2p, dis)

    return out[:, :c]


# speedup vs baseline: 5.1836x; 1.0239x over previous
import jax
import jax.numpy as jnp
from jax.experimental import pallas as pl
from jax.experimental.pallas import tpu as pltpu

_T = 256
_CH = 512


def _build_a_kernel(bounds_ref, rl_ref, cl_ref, at_ref, deg_ref, acc_ref):
    i = pl.program_id(0)
    n = at_ref.shape[0]
    acc_ref[...] = jnp.zeros_like(acc_ref)
    k0 = bounds_ref[i] // _CH
    k1 = (bounds_ref[i + 1] + _CH - 1) // _CH

    row_iota = jax.lax.broadcasted_iota(jnp.int32, (_T, _CH), 0)
    col_iota = jax.lax.broadcasted_iota(jnp.int32, (n, _CH), 0)

    def body(k, _):
        rv = rl_ref[pl.ds(k, 1), :] - i * _T
        cv = cl_ref[pl.ds(k, 1), :]
        d_oh = (row_iota == rv).astype(jnp.bfloat16)
        s_oh = (col_iota == cv).astype(jnp.bfloat16)
        acc_ref[...] = (acc_ref[...] + jax.lax.dot_general(
            s_oh, d_oh, (((1,), (1,)), ((), ())),
            preferred_element_type=jnp.float32)).astype(jnp.bfloat16)
        return _

    jax.lax.fori_loop(k0, k1, body, 0)
    acc = acc_ref[...]
    at_ref[...] = acc
    deg_ref[...] = jnp.sum(acc.astype(jnp.float32), axis=0, keepdims=True)


def _xw_kernel(x_ref, w_ref, o_ref):
    x = x_ref[...].astype(jnp.bfloat16)
    o_ref[...] = jnp.dot(
        x, w_ref[...], preferred_element_type=jnp.float32
    ).astype(jnp.bfloat16)


def _h_kernel(at_ref, p_ref, w2_ref, b1_ref, dis_ref, q_ref):
    i = pl.program_id(0)
    h = jax.lax.dot_general(
        at_ref[...], p_ref[...], (((0,), (0,)), ((), ())),
        preferred_element_type=jnp.float32)
    p_i = p_ref[pl.ds(i * _T, _T), :].astype(jnp.float32)
    dis_i = dis_ref[pl.ds(i * _T, _T), :]
    h = jnp.maximum(dis_i * (h + p_i) + b1_ref[...], 0.0)
    q = jnp.dot(h.astype(jnp.bfloat16), w2_ref[...],
                preferred_element_type=jnp.float32)
    q_ref[...] = (dis_i * q).astype(jnp.bfloat16)


def _out_kernel(at_ref, q_ref, b2_ref, dis_ref, o_ref):
    i = pl.program_id(0)
    o = jax.lax.dot_general(
        at_ref[...], q_ref[...], (((0,), (0,)), ((), ())),
        preferred_element_type=jnp.float32)
    q_i = q_ref[pl.ds(i * _T, _T), :].astype(jnp.float32)
    dis_i = dis_ref[pl.ds(i * _T, _T), :]
    c = o_ref.shape[1]
    o_ref[...] = (dis_i * (o + q_i) + b2_ref[...])[:, :c]


def kernel(x, edge_index, w1, b1, w2, b2):
    n, f_in = x.shape
    hidden = w1.shape[1]
    c = w2.shape[1]
    c_pad = max(128, ((c + 127) // 128) * 128)
    e = edge_index.shape[1]
    n_tiles = n // _T
    e_pad = ((e + _CH - 1) // _CH) * _CH

    src = edge_index[0]
    dst = edge_index[1]
    lin = jnp.sort(jnp.concatenate(
        [dst * n + src, jnp.full((e_pad - e,), n * n, jnp.int32)]))
    tile_of_edge = dst // _T
    bounds = jnp.concatenate([
        jnp.zeros((1,), jnp.int32),
        jnp.cumsum(jnp.sum(
            tile_of_edge[None, :] == jnp.arange(n_tiles)[:, None],
            axis=1, dtype=jnp.int32))])
    rl = (lin // n).reshape(e_pad // _CH, _CH)
    cl = (lin % n).reshape(e_pad // _CH, _CH)

    w1b = w1.astype(jnp.bfloat16)
    w2p = jnp.zeros((hidden, c_pad), jnp.bfloat16).at[:, :c].set(
        w2.astype(jnp.bfloat16))
    b2p = jnp.zeros((1, c_pad), jnp.float32).at[:, :c].set(b2)

    grid = (n_tiles,)
    par = pltpu.CompilerParams(dimension_semantics=("parallel",))

    a_t, deg = pl.pallas_call(
        _build_a_kernel,
        out_shape=(jax.ShapeDtypeStruct((n, n), jnp.bfloat16),
                   jax.ShapeDtypeStruct((1, n), jnp.float32)),
        grid=grid,
        in_specs=[
            pl.BlockSpec(memory_space=pltpu.SMEM),
            pl.BlockSpec((e_pad // _CH, _CH), lambda i: (0, 0)),
            pl.BlockSpec((e_pad // _CH, _CH), lambda i: (0, 0)),
        ],
        out_specs=(pl.BlockSpec((n, _T), lambda i: (0, i)),
                   pl.BlockSpec((1, _T), lambda i: (0, i))),
        scratch_shapes=[pltpu.VMEM((n, _T), jnp.bfloat16)],
        compiler_params=par,
    )(bounds, rl, cl)

    dis = (1.0 / jnp.sqrt(deg + 1.0)).reshape(n, 1)

    p = pl.pallas_call(
        _xw_kernel,
        out_shape=jax.ShapeDtypeStruct((n, hidden), jnp.bfloat16),
        grid=grid,
        in_specs=[
            pl.BlockSpec((_T, f_in), lambda i: (i, 0)),
            pl.BlockSpec((f_in, hidden), lambda i: (0, 0)),
        ],
        out_specs=pl.BlockSpec((_T, hidden), lambda i: (i, 0)),
        compiler_params=par,
    )(x, w1b)

    p_s = (p.astype(jnp.float32) * dis).astype(jnp.bfloat16)

    q = pl.pallas_call(
        _h_kernel,
        out_shape=jax.ShapeDtypeStruct((n, c_pad), jnp.bfloat16),
        grid=grid,
        in_specs=[
            pl.BlockSpec((n, _T), lambda i: (0, i)),
            pl.BlockSpec((n, hidden), lambda i: (0, 0)),
            pl.BlockSpec((hidden, c_pad), lambda i: (0, 0)),
            pl.BlockSpec((1, hidden), lambda i: (0, 0)),
            pl.BlockSpec((n, 1), lambda i: (0, 0)),
        ],
        out_specs=pl.BlockSpec((_T, c_pad), lambda i: (i, 0)),
        compiler_params=par,
    )(a_t, p_s, w2p, b1, dis)

    out = pl.pallas_call(
        _out_kernel,
        out_shape=jax.ShapeDtypeStruct((n, c), jnp.float32),
        grid=grid,
        in_specs=[
            pl.BlockSpec((n, _T), lambda i: (0, i)),
            pl.BlockSpec((n, c_pad), lambda i: (0, 0)),
            pl.BlockSpec((1, c_pad), lambda i: (0, 0)),
            pl.BlockSpec((n, 1), lambda i: (0, 0)),
        ],
        out_specs=pl.BlockSpec((_T, c), lambda i: (i, 0)),
        compiler_params=par,
    )(a_t, q, b2p, dis)

    return out


# speedup vs baseline: 5.4645x; 1.0542x over previous
import jax
import jax.numpy as jnp
from jax.experimental import pallas as pl
from jax.experimental.pallas import tpu as pltpu

_T = 256
_CH = 512


def _build_a_kernel(bounds_ref, rl_ref, cl_ref, at_ref, deg_ref):
    i = pl.program_id(0)
    n = at_ref.shape[0]
    half = n // 2
    k0 = bounds_ref[i] // _CH
    k1 = (bounds_ref[i + 1] + _CH - 1) // _CH

    row_iota = jax.lax.broadcasted_iota(jnp.int32, (_T, _CH), 0)
    col_iota = jax.lax.broadcasted_iota(jnp.int32, (n, _CH), 0)

    junk = jnp.concatenate(
        [pltpu.matmul_pop(0, (half, _T), jnp.float32, m) for m in (0, 1)],
        axis=0)
    at_ref[...] = junk.astype(jnp.bfloat16)

    def body(k, _):
        rv = rl_ref[pl.ds(k, 1), :] - i * _T
        cv = cl_ref[pl.ds(k, 1), :]
        d_oh = (row_iota == rv).astype(jnp.bfloat16)
        s_oh = (col_iota == cv).astype(jnp.bfloat16)
        for kt in range(_CH // 256):
            d_t = d_oh[:, kt * 256:(kt + 1) * 256]
            s_t = s_oh[:, kt * 256:(kt + 1) * 256]
            for m in (0, 1):
                pltpu.matmul_push_rhs(d_t, staging_register=0, mxu_index=m,
                                      transpose=True)
                pltpu.matmul_acc_lhs(0, s_t[m * half:(m + 1) * half, :],
                                     mxu_index=m, load_staged_rhs=0)
        return _

    jax.lax.fori_loop(k0, k1, body, 0)
    acc = jnp.concatenate(
        [pltpu.matmul_pop(0, (half, _T), jnp.float32, m) for m in (0, 1)],
        axis=0)
    at_ref[...] = acc.astype(jnp.bfloat16)
    deg_ref[...] = jnp.sum(acc, axis=0, keepdims=True)


def _xw_kernel(x_ref, w_ref, o_ref):
    x = x_ref[...].astype(jnp.bfloat16)
    o_ref[...] = jnp.dot(
        x, w_ref[...], preferred_element_type=jnp.float32
    ).astype(jnp.bfloat16)


def _h_kernel(at_ref, p_ref, w2_ref, b1_ref, dis_ref, q_ref):
    i = pl.program_id(0)
    h = jax.lax.dot_general(
        at_ref[...], p_ref[...], (((0,), (0,)), ((), ())),
        preferred_element_type=jnp.float32)
    p_i = p_ref[pl.ds(i * _T, _T), :].astype(jnp.float32)
    dis_i = dis_ref[pl.ds(i * _T, _T), :]
    h = jnp.maximum(dis_i * (h + p_i) + b1_ref[...], 0.0)
    q = jnp.dot(h.astype(jnp.bfloat16), w2_ref[...],
                preferred_element_type=jnp.float32)
    q_ref[...] = (dis_i * q).astype(jnp.bfloat16)


def _out_kernel(at_ref, q_ref, b2_ref, dis_ref, o_ref):
    i = pl.program_id(0)
    o = jax.lax.dot_general(
        at_ref[...], q_ref[...], (((0,), (0,)), ((), ())),
        preferred_element_type=jnp.float32)
    q_i = q_ref[pl.ds(i * _T, _T), :].astype(jnp.float32)
    dis_i = dis_ref[pl.ds(i * _T, _T), :]
    c = o_ref.shape[1]
    o_ref[...] = (dis_i * (o + q_i) + b2_ref[...])[:, :c]


def kernel(x, edge_index, w1, b1, w2, b2):
    n, f_in = x.shape
    hidden = w1.shape[1]
    c = w2.shape[1]
    c_pad = max(128, ((c + 127) // 128) * 128)
    e = edge_index.shape[1]
    n_tiles = n // _T
    e_pad = ((e + _CH - 1) // _CH) * _CH

    src = edge_index[0]
    dst = edge_index[1]
    lin = jnp.sort(jnp.concatenate(
        [dst * n + src, jnp.full((e_pad - e,), n * n, jnp.int32)]))
    tile_of_edge = dst // _T
    bounds = jnp.concatenate([
        jnp.zeros((1,), jnp.int32),
        jnp.cumsum(jnp.sum(
            tile_of_edge[None, :] == jnp.arange(n_tiles)[:, None],
            axis=1, dtype=jnp.int32))])
    rl = (lin // n).reshape(e_pad // _CH, _CH)
    cl = (lin % n).reshape(e_pad // _CH, _CH)

    w1b = w1.astype(jnp.bfloat16)
    w2p = jnp.zeros((hidden, c_pad), jnp.bfloat16).at[:, :c].set(
        w2.astype(jnp.bfloat16))
    b2p = jnp.zeros((1, c_pad), jnp.float32).at[:, :c].set(b2)

    grid = (n_tiles,)
    par = pltpu.CompilerParams(dimension_semantics=("parallel",))

    a_t, deg = pl.pallas_call(
        _build_a_kernel,
        out_shape=(jax.ShapeDtypeStruct((n, n), jnp.bfloat16),
                   jax.ShapeDtypeStruct((1, n), jnp.float32)),
        grid=grid,
        in_specs=[
            pl.BlockSpec(memory_space=pltpu.SMEM),
            pl.BlockSpec((e_pad // _CH, _CH), lambda i: (0, 0)),
            pl.BlockSpec((e_pad // _CH, _CH), lambda i: (0, 0)),
        ],
        out_specs=(pl.BlockSpec((n, _T), lambda i: (0, i)),
                   pl.BlockSpec((1, _T), lambda i: (0, i))),
        compiler_params=par,
    )(bounds, rl, cl)

    dis = (1.0 / jnp.sqrt(deg + 1.0)).reshape(n, 1)

    p = pl.pallas_call(
        _xw_kernel,
        out_shape=jax.ShapeDtypeStruct((n, hidden), jnp.bfloat16),
        grid=grid,
        in_specs=[
            pl.BlockSpec((_T, f_in), lambda i: (i, 0)),
            pl.BlockSpec((f_in, hidden), lambda i: (0, 0)),
        ],
        out_specs=pl.BlockSpec((_T, hidden), lambda i: (i, 0)),
        compiler_params=par,
    )(x, w1b)

    p_s = (p.astype(jnp.float32) * dis).astype(jnp.bfloat16)

    q = pl.pallas_call(
        _h_kernel,
        out_shape=jax.ShapeDtypeStruct((n, c_pad), jnp.bfloat16),
        grid=grid,
        in_specs=[
            pl.BlockSpec((n, _T), lambda i: (0, i)),
            pl.BlockSpec((n, hidden), lambda i: (0, 0)),
            pl.BlockSpec((hidden, c_pad), lambda i: (0, 0)),
            pl.BlockSpec((1, hidden), lambda i: (0, 0)),
            pl.BlockSpec((n, 1), lambda i: (0, 0)),
        ],
        out_specs=pl.BlockSpec((_T, c_pad), lambda i: (i, 0)),
        compiler_params=par,
    )(a_t, p_s, w2p, b1, dis)

    out = pl.pallas_call(
        _out_kernel,
        out_shape=jax.ShapeDtypeStruct((n, c), jnp.float32),
        grid=grid,
        in_specs=[
            pl.BlockSpec((n, _T), lambda i: (0, i)),
            pl.BlockSpec((n, c_pad), lambda i: (0, 0)),
            pl.BlockSpec((1, c_pad), lambda i: (0, 0)),
            pl.BlockSpec((n, 1), lambda i: (0, 0)),
        ],
        out_specs=pl.BlockSpec((_T, c), lambda i: (i, 0)),
        compiler_params=par,
    )(a_t, q, b2p, dis)

    return out


# speedup vs baseline: 5.5047x; 1.0073x over previous
import jax
import jax.numpy as jnp
from jax.experimental import pallas as pl
from jax.experimental.pallas import tpu as pltpu

_T = 256
_CH = 512


def _build_a_kernel(bounds_ref, rl_ref, cl_ref, at_ref, deg_ref):
    i = pl.program_id(0)
    n = at_ref.shape[0]
    half = n // 2
    k0 = bounds_ref[i] // _CH
    k1 = (bounds_ref[i + 1] + _CH - 1) // _CH

    row_iota = jax.lax.broadcasted_iota(jnp.int32, (_T, _CH), 0)
    col_iota = jax.lax.broadcasted_iota(jnp.int32, (n, _CH), 0)

    junk = jnp.concatenate(
        [pltpu.matmul_pop(0, (half, _T), jnp.float32, m) for m in (0, 1)],
        axis=0)
    at_ref[...] = junk.astype(jnp.bfloat16)

    def do_chunk(k):
        rv = rl_ref[pl.ds(k, 1), :] - i * _T
        cv = cl_ref[pl.ds(k, 1), :]
        d_oh = (row_iota == rv).astype(jnp.bfloat16)
        s_oh = (col_iota == cv).astype(jnp.bfloat16)
        for kt in range(_CH // 256):
            d_t = d_oh[:, kt * 256:(kt + 1) * 256]
            s_t = s_oh[:, kt * 256:(kt + 1) * 256]
            for m in (0, 1):
                pltpu.matmul_push_rhs(d_t, staging_register=0, mxu_index=m,
                                      transpose=True)
                pltpu.matmul_acc_lhs(0, s_t[m * half:(m + 1) * half, :],
                                     mxu_index=m, load_staged_rhs=0)

    def body(p_idx, _):
        do_chunk(2 * p_idx)
        do_chunk(2 * p_idx + 1)
        return _

    jax.lax.fori_loop(k0 // 2, (k1 + 1) // 2, body, 0)
    acc = jnp.concatenate(
        [pltpu.matmul_pop(0, (half, _T), jnp.float32, m) for m in (0, 1)],
        axis=0)
    at_ref[...] = acc.astype(jnp.bfloat16)
    deg_ref[...] = jnp.sum(acc, axis=0, keepdims=True)


def _xw_kernel(x_ref, w_ref, dis_ref, o_ref):
    i = pl.program_id(0)
    x = x_ref[...].astype(jnp.bfloat16)
    dis_i = dis_ref[pl.ds(i * _T, _T), :]
    o_ref[...] = (dis_i * jnp.dot(
        x, w_ref[...], preferred_element_type=jnp.float32
    )).astype(jnp.bfloat16)


def _h_kernel(at_ref, p_ref, w2_ref, b1_ref, dis_ref, q_ref):
    i = pl.program_id(0)
    h = jax.lax.dot_general(
        at_ref[...], p_ref[...], (((0,), (0,)), ((), ())),
        preferred_element_type=jnp.float32)
    p_i = p_ref[pl.ds(i * _T, _T), :].astype(jnp.float32)
    dis_i = dis_ref[pl.ds(i * _T, _T), :]
    h = jnp.maximum(dis_i * (h + p_i) + b1_ref[...], 0.0)
    q = jnp.dot(h.astype(jnp.bfloat16), w2_ref[...],
                preferred_element_type=jnp.float32)
    q_ref[...] = (dis_i * q).astype(jnp.bfloat16)


def _out_kernel(at_ref, q_ref, b2_ref, dis_ref, o_ref):
    i = pl.program_id(0)
    o = jax.lax.dot_general(
        at_ref[...], q_ref[...], (((0,), (0,)), ((), ())),
        preferred_element_type=jnp.float32)
    q_i = q_ref[pl.ds(i * _T, _T), :].astype(jnp.float32)
    dis_i = dis_ref[pl.ds(i * _T, _T), :]
    c = o_ref.shape[1]
    o_ref[...] = (dis_i * (o + q_i) + b2_ref[...])[:, :c]


def kernel(x, edge_index, w1, b1, w2, b2):
    n, f_in = x.shape
    hidden = w1.shape[1]
    c = w2.shape[1]
    c_pad = max(128, ((c + 127) // 128) * 128)
    e = edge_index.shape[1]
    n_tiles = n // _T
    e_pad = ((e + _CH - 1) // _CH) * _CH

    src = edge_index[0]
    dst = edge_index[1]
    lin = jnp.sort(jnp.concatenate(
        [dst * n + src, jnp.full((e_pad - e,), n * n, jnp.int32)]))
    tile_of_edge = dst // _T
    bounds = jnp.concatenate([
        jnp.zeros((1,), jnp.int32),
        jnp.cumsum(jnp.sum(
            tile_of_edge[None, :] == jnp.arange(n_tiles)[:, None],
            axis=1, dtype=jnp.int32))])
    rl = (lin // n).reshape(e_pad // _CH, _CH)
    cl = (lin % n).reshape(e_pad // _CH, _CH)

    w1b = w1.astype(jnp.bfloat16)
    w2p = jnp.zeros((hidden, c_pad), jnp.bfloat16).at[:, :c].set(
        w2.astype(jnp.bfloat16))
    b2p = jnp.zeros((1, c_pad), jnp.float32).at[:, :c].set(b2)

    grid = (n_tiles,)
    par = pltpu.CompilerParams(dimension_semantics=("parallel",))

    a_t, deg = pl.pallas_call(
        _build_a_kernel,
        out_shape=(jax.ShapeDtypeStruct((n, n), jnp.bfloat16),
                   jax.ShapeDtypeStruct((1, n), jnp.float32)),
        grid=grid,
        in_specs=[
            pl.BlockSpec(memory_space=pltpu.SMEM),
            pl.BlockSpec((e_pad // _CH, _CH), lambda i: (0, 0)),
            pl.BlockSpec((e_pad // _CH, _CH), lambda i: (0, 0)),
        ],
        out_specs=(pl.BlockSpec((n, _T), lambda i: (0, i)),
                   pl.BlockSpec((1, _T), lambda i: (0, i))),
        compiler_params=par,
    )(bounds, rl, cl)

    dis = (1.0 / jnp.sqrt(deg + 1.0)).reshape(n, 1)

    p_s = pl.pallas_call(
        _xw_kernel,
        out_shape=jax.ShapeDtypeStruct((n, hidden), jnp.bfloat16),
        grid=grid,
        in_specs=[
            pl.BlockSpec((_T, f_in), lambda i: (i, 0)),
            pl.BlockSpec((f_in, hidden), lambda i: (0, 0)),
            pl.BlockSpec((n, 1), lambda i: (0, 0)),
        ],
        out_specs=pl.BlockSpec((_T, hidden), lambda i: (i, 0)),
        compiler_params=par,
    )(x, w1b, dis)

    q = pl.pallas_call(
        _h_kernel,
        out_shape=jax.ShapeDtypeStruct((n, c_pad), jnp.bfloat16),
        grid=grid,
        in_specs=[
            pl.BlockSpec((n, _T), lambda i: (0, i)),
            pl.BlockSpec((n, hidden), lambda i: (0, 0)),
            pl.BlockSpec((hidden, c_pad), lambda i: (0, 0)),
            pl.BlockSpec((1, hidden), lambda i: (0, 0)),
            pl.BlockSpec((n, 1), lambda i: (0, 0)),
        ],
        out_specs=pl.BlockSpec((_T, c_pad), lambda i: (i, 0)),
        compiler_params=par,
    )(a_t, p_s, w2p, b1, dis)

    out = pl.pallas_call(
        _out_kernel,
        out_shape=jax.ShapeDtypeStruct((n, c), jnp.float32),
        grid=grid,
        in_specs=[
            pl.BlockSpec((n, _T), lambda i: (0, i)),
            pl.BlockSpec((n, c_pad), lambda i: (0, 0)),
            pl.BlockSpec((1, c_pad), lambda i: (0, 0)),
            pl.BlockSpec((n, 1), lambda i: (0, 0)),
        ],
        out_specs=pl.BlockSpec((_T, c), lambda i: (i, 0)),
        compiler_params=par,
    )(a_t, q, b2p, dis)

    return out


# speedup vs baseline: 5.8561x; 1.0638x over previous
import jax
import jax.numpy as jnp
from jax.experimental import pallas as pl
from jax.experimental.pallas import tpu as pltpu

_T = 256
_TB = 512
_CH = 512


def _build_a_kernel(bounds_ref, rl_ref, cl_ref, at_ref, deg_ref):
    i = pl.program_id(0)
    n = at_ref.shape[0]
    half = n // 2
    k0 = bounds_ref[i] // _CH
    k1 = (bounds_ref[i + 1] + _CH - 1) // _CH

    row_iota = jax.lax.broadcasted_iota(jnp.int32, (_T, _CH), 0)
    col_iota = jax.lax.broadcasted_iota(jnp.int32, (n, _CH), 0)

    junk = jnp.concatenate(
        [pltpu.matmul_pop(0, (half, _T), jnp.float32, m) for m in (0, 1)],
        axis=0)
    at_ref[...] = junk.astype(jnp.bfloat16)

    def do_chunk(k):
        rv = rl_ref[pl.ds(k, 1), :] - i * _T
        cv = cl_ref[pl.ds(k, 1), :]
        d_oh = (row_iota == rv).astype(jnp.bfloat16)
        s_oh = (col_iota == cv).astype(jnp.bfloat16)
        for kt in range(_CH // 256):
            d_t = d_oh[:, kt * 256:(kt + 1) * 256]
            s_t = s_oh[:, kt * 256:(kt + 1) * 256]
            for m in (0, 1):
                pltpu.matmul_push_rhs(d_t, staging_register=0, mxu_index=m,
                                      transpose=True)
                pltpu.matmul_acc_lhs(0, s_t[m * half:(m + 1) * half, :],
                                     mxu_index=m, load_staged_rhs=0)

    def body(p_idx, _):
        do_chunk(2 * p_idx)
        do_chunk(2 * p_idx + 1)
        return _

    jax.lax.fori_loop(k0 // 2, (k1 + 1) // 2, body, 0)
    acc = jnp.concatenate(
        [pltpu.matmul_pop(0, (half, _T), jnp.float32, m) for m in (0, 1)],
        axis=0)
    at_ref[...] = acc.astype(jnp.bfloat16)
    deg_ref[...] = jnp.sum(acc, axis=0, keepdims=True)


def _xw_kernel(x_ref, w_ref, dis_ref, o_ref):
    i = pl.program_id(0)
    x = x_ref[...].astype(jnp.bfloat16)
    w = w_ref[...].astype(jnp.bfloat16)
    dis_i = dis_ref[pl.ds(i * _T, _T), :]
    o_ref[...] = (dis_i * jnp.dot(
        x, w, preferred_element_type=jnp.float32
    )).astype(jnp.bfloat16)


def _h_kernel(at_ref, p_ref, w2_ref, b1_ref, dis_ref, q_ref):
    i = pl.program_id(0)
    h = jax.lax.dot_general(
        at_ref[...], p_ref[...], (((0,), (0,)), ((), ())),
        preferred_element_type=jnp.float32)
    p_i = p_ref[pl.ds(i * _TB, _TB), :].astype(jnp.float32)
    dis_i = dis_ref[pl.ds(i * _TB, _TB), :]
    h = jnp.maximum(dis_i * (h + p_i) + b1_ref[...], 0.0)
    q = jnp.dot(h.astype(jnp.bfloat16), w2_ref[...],
                preferred_element_type=jnp.float32)
    q_ref[...] = (dis_i * q).astype(jnp.bfloat16)


def _out_kernel(at_ref, q_ref, b2_ref, dis_ref, o_ref):
    i = pl.program_id(0)
    o = jax.lax.dot_general(
        at_ref[...], q_ref[...], (((0,), (0,)), ((), ())),
        preferred_element_type=jnp.float32)
    q_i = q_ref[pl.ds(i * _TB, _TB), :].astype(jnp.float32)
    dis_i = dis_ref[pl.ds(i * _TB, _TB), :]
    c = o_ref.shape[1]
    o_ref[...] = (dis_i * (o + q_i) + b2_ref[...])[:, :c]


def kernel(x, edge_index, w1, b1, w2, b2):
    n, f_in = x.shape
    hidden = w1.shape[1]
    c = w2.shape[1]
    c_pad = max(128, ((c + 127) // 128) * 128)
    e = edge_index.shape[1]
    n_tiles = n // _T
    e_pad = ((e + _CH - 1) // _CH) * _CH

    src = edge_index[0]
    dst = edge_index[1]
    lin = jnp.sort(jnp.concatenate(
        [dst * n + src, jnp.full((e_pad - e,), n * n, jnp.int32)]))
    tile_of_edge = dst // _T
    bounds = jnp.concatenate([
        jnp.zeros((1,), jnp.int32),
        jnp.cumsum(jnp.sum(
            tile_of_edge[None, :] == jnp.arange(n_tiles)[:, None],
            axis=1, dtype=jnp.int32))])
    rl = (lin // n).reshape(e_pad // _CH, _CH)
    cl = (lin % n).reshape(e_pad // _CH, _CH)

    w2p = jnp.zeros((hidden, c_pad), jnp.bfloat16).at[:, :c].set(
        w2.astype(jnp.bfloat16))
    b2p = jnp.zeros((1, c_pad), jnp.float32).at[:, :c].set(b2)

    grid = (n_tiles,)
    par = pltpu.CompilerParams(dimension_semantics=("parallel",))

    a_t, deg = pl.pallas_call(
        _build_a_kernel,
        out_shape=(jax.ShapeDtypeStruct((n, n), jnp.bfloat16),
                   jax.ShapeDtypeStruct((1, n), jnp.float32)),
        grid=grid,
        in_specs=[
            pl.BlockSpec(memory_space=pltpu.SMEM),
            pl.BlockSpec((e_pad // _CH, _CH), lambda i: (0, 0)),
            pl.BlockSpec((e_pad // _CH, _CH), lambda i: (0, 0)),
        ],
        out_specs=(pl.BlockSpec((n, _T), lambda i: (0, i)),
                   pl.BlockSpec((1, _T), lambda i: (0, i))),
        compiler_params=par,
    )(bounds, rl, cl)

    dis = (1.0 / jnp.sqrt(deg + 1.0)).reshape(n, 1)

    p_s = pl.pallas_call(
        _xw_kernel,
        out_shape=jax.ShapeDtypeStruct((n, hidden), jnp.bfloat16),
        grid=grid,
        in_specs=[
            pl.BlockSpec((_T, f_in), lambda i: (i, 0)),
            pl.BlockSpec((f_in, hidden), lambda i: (0, 0)),
            pl.BlockSpec((n, 1), lambda i: (0, 0)),
        ],
        out_specs=pl.BlockSpec((_T, hidden), lambda i: (i, 0)),
        compiler_params=par,
    )(x, w1, dis)

    grid_b = (n // _TB,)
    q = pl.pallas_call(
        _h_kernel,
        out_shape=jax.ShapeDtypeStruct((n, c_pad), jnp.bfloat16),
        grid=grid_b,
        in_specs=[
            pl.BlockSpec((n, _TB), lambda i: (0, i)),
            pl.BlockSpec((n, hidden), lambda i: (0, 0)),
            pl.BlockSpec((hidden, c_pad), lambda i: (0, 0)),
            pl.BlockSpec((1, hidden), lambda i: (0, 0)),
            pl.BlockSpec((n, 1), lambda i: (0, 0)),
        ],
        out_specs=pl.BlockSpec((_TB, c_pad), lambda i: (i, 0)),
        compiler_params=par,
    )(a_t, p_s, w2p, b1, dis)

    out = pl.pallas_call(
        _out_kernel,
        out_shape=jax.ShapeDtypeStruct((n, c), jnp.float32),
        grid=grid_b,
        in_specs=[
            pl.BlockSpec((n, _TB), lambda i: (0, i)),
            pl.BlockSpec((n, c_pad), lambda i: (0, 0)),
            pl.BlockSpec((1, c_pad), lambda i: (0, 0)),
            pl.BlockSpec((n, 1), lambda i: (0, 0)),
        ],
        out_specs=pl.BlockSpec((_TB, c), lambda i: (i, 0)),
        compiler_params=par,
    )(a_t, q, b2p, dis)

    return out


# speedup vs baseline: 6.3649x; 1.0869x over previous
import jax
import jax.numpy as jnp
from jax.experimental import pallas as pl
from jax.experimental.pallas import tpu as pltpu

_T = 256
_TB = 512
_CH = 512


def _build_a_kernel(bounds_ref, rl_ref, cl_ref, at_ref, deg_ref):
    i = pl.program_id(0)
    n = at_ref.shape[0]
    half = n // 2
    k0 = bounds_ref[i] // _CH
    k1 = (bounds_ref[i + 1] + _CH - 1) // _CH

    row_iota = jax.lax.broadcasted_iota(jnp.int32, (_T, _CH), 0)
    col_iota = jax.lax.broadcasted_iota(jnp.int32, (n, _CH), 0)

    junk = jnp.concatenate(
        [pltpu.matmul_pop(0, (half, _T), jnp.float32, m) for m in (0, 1)],
        axis=0)
    at_ref[...] = junk.astype(jnp.bfloat16)

    def do_chunk(k):
        rv = rl_ref[pl.ds(k, 1), :] - i * _T
        cv = cl_ref[pl.ds(k, 1), :]
        d_oh = (row_iota == rv).astype(jnp.float8_e4m3fn)
        s_oh = (col_iota == cv).astype(jnp.float8_e4m3fn)
        for kt in range(_CH // 256):
            d_t = d_oh[:, kt * 256:(kt + 1) * 256]
            s_t = s_oh[:, kt * 256:(kt + 1) * 256]
            for m in (0, 1):
                pltpu.matmul_push_rhs(d_t, staging_register=0, mxu_index=m,
                                      transpose=True)
                pltpu.matmul_acc_lhs(0, s_t[m * half:(m + 1) * half, :],
                                     mxu_index=m, load_staged_rhs=0)

    def body(p_idx, _):
        do_chunk(2 * p_idx)
        do_chunk(2 * p_idx + 1)
        return _

    jax.lax.fori_loop(k0 // 2, (k1 + 1) // 2, body, 0)
    acc = jnp.concatenate(
        [pltpu.matmul_pop(0, (half, _T), jnp.float32, m) for m in (0, 1)],
        axis=0)
    at_ref[...] = acc.astype(jnp.bfloat16)
    deg_ref[...] = jnp.sum(acc, axis=0, keepdims=True)


def _xw_kernel(x_ref, w_ref, dis_ref, o_ref):
    i = pl.program_id(0)
    x = x_ref[...].astype(jnp.bfloat16)
    w = w_ref[...].astype(jnp.bfloat16)
    dis_i = dis_ref[pl.ds(i * _T, _T), :]
    o_ref[...] = (dis_i * jnp.dot(
        x, w, preferred_element_type=jnp.float32
    )).astype(jnp.bfloat16)


def _h_kernel(at_ref, p_ref, w2_ref, b1_ref, dis_ref, q_ref):
    i = pl.program_id(0)
    h = jax.lax.dot_general(
        at_ref[...], p_ref[...], (((0,), (0,)), ((), ())),
        preferred_element_type=jnp.float32)
    p_i = p_ref[pl.ds(i * _TB, _TB), :].astype(jnp.float32)
    dis_i = dis_ref[pl.ds(i * _TB, _TB), :]
    h = jnp.maximum(dis_i * (h + p_i) + b1_ref[...], 0.0)
    q = jnp.dot(h.astype(jnp.bfloat16), w2_ref[...],
                preferred_element_type=jnp.float32)
    q_ref[...] = (dis_i * q).astype(jnp.bfloat16)


def _out_kernel(at_ref, q_ref, b2_ref, dis_ref, o_ref):
    i = pl.program_id(0)
    o = jax.lax.dot_general(
        at_ref[...], q_ref[...], (((0,), (0,)), ((), ())),
        preferred_element_type=jnp.float32)
    q_i = q_ref[pl.ds(i * _TB, _TB), :].astype(jnp.float32)
    dis_i = dis_ref[pl.ds(i * _TB, _TB), :]
    c = o_ref.shape[1]
    o_ref[...] = (dis_i * (o + q_i) + b2_ref[...])[:, :c]


def kernel(x, edge_index, w1, b1, w2, b2):
    n, f_in = x.shape
    hidden = w1.shape[1]
    c = w2.shape[1]
    c_pad = max(128, ((c + 127) // 128) * 128)
    e = edge_index.shape[1]
    n_tiles = n // _T
    e_pad = ((e + _CH - 1) // _CH) * _CH

    src = edge_index[0]
    dst = edge_index[1]
    lin = jnp.sort(jnp.concatenate(
        [dst * n + src, jnp.full((e_pad - e,), n * n, jnp.int32)]))
    tile_of_edge = dst // _T
    bounds = jnp.concatenate([
        jnp.zeros((1,), jnp.int32),
        jnp.cumsum(jnp.sum(
            tile_of_edge[None, :] == jnp.arange(n_tiles)[:, None],
            axis=1, dtype=jnp.int32))])
    rl = (lin // n).reshape(e_pad // _CH, _CH)
    cl = (lin % n).reshape(e_pad // _CH, _CH)

    w2p = jnp.zeros((hidden, c_pad), jnp.bfloat16).at[:, :c].set(
        w2.astype(jnp.bfloat16))
    b2p = jnp.zeros((1, c_pad), jnp.float32).at[:, :c].set(b2)

    grid = (n_tiles,)
    par = pltpu.CompilerParams(dimension_semantics=("parallel",))

    a_t, deg = pl.pallas_call(
        _build_a_kernel,
        out_shape=(jax.ShapeDtypeStruct((n, n), jnp.bfloat16),
                   jax.ShapeDtypeStruct((1, n), jnp.float32)),
        grid=grid,
        in_specs=[
            pl.BlockSpec(memory_space=pltpu.SMEM),
            pl.BlockSpec((e_pad // _CH, _CH), lambda i: (0, 0)),
            pl.BlockSpec((e_pad // _CH, _CH), lambda i: (0, 0)),
        ],
        out_specs=(pl.BlockSpec((n, _T), lambda i: (0, i)),
                   pl.BlockSpec((1, _T), lambda i: (0, i))),
        compiler_params=par,
    )(bounds, rl, cl)

    dis = (1.0 / jnp.sqrt(deg + 1.0)).reshape(n, 1)

    p_s = pl.pallas_call(
        _xw_kernel,
        out_shape=jax.ShapeDtypeStruct((n, hidden), jnp.bfloat16),
        grid=grid,
        in_specs=[
            pl.BlockSpec((_T, f_in), lambda i: (i, 0)),
            pl.BlockSpec((f_in, hidden), lambda i: (0, 0)),
            pl.BlockSpec((n, 1), lambda i: (0, 0)),
        ],
        out_specs=pl.BlockSpec((_T, hidden), lambda i: (i, 0)),
        compiler_params=par,
    )(x, w1, dis)

    grid_b = (n // _TB,)
    q = pl.pallas_call(
        _h_kernel,
        out_shape=jax.ShapeDtypeStruct((n, c_pad), jnp.bfloat16),
        grid=grid_b,
        in_specs=[
            pl.BlockSpec((n, _TB), lambda i: (0, i)),
            pl.BlockSpec((n, hidden), lambda i: (0, 0)),
            pl.BlockSpec((hidden, c_pad), lambda i: (0, 0)),
            pl.BlockSpec((1, hidden), lambda i: (0, 0)),
            pl.BlockSpec((n, 1), lambda i: (0, 0)),
        ],
        out_specs=pl.BlockSpec((_TB, c_pad), lambda i: (i, 0)),
        compiler_params=par,
    )(a_t, p_s, w2p, b1, dis)

    out = pl.pallas_call(
        _out_kernel,
        out_shape=jax.ShapeDtypeStruct((n, c), jnp.float32),
        grid=grid_b,
        in_specs=[
            pl.BlockSpec((n, _TB), lambda i: (0, i)),
            pl.BlockSpec((n, c_pad), lambda i: (0, 0)),
            pl.BlockSpec((1, c_pad), lambda i: (0, 0)),
            pl.BlockSpec((n, 1), lambda i: (0, 0)),
        ],
        out_specs=pl.BlockSpec((_TB, c), lambda i: (i, 0)),
        compiler_params=par,
    )(a_t, q, b2p, dis)

    return out


# speedup vs baseline: 6.5027x; 1.0217x over previous
import jax
import jax.numpy as jnp
from jax.experimental import pallas as pl
from jax.experimental.pallas import tpu as pltpu

_T = 256
_TB = 512
_CH = 512


def _build_a_kernel(bounds_ref, rl_ref, cl_ref, at_ref, deg_ref):
    i = pl.program_id(0)
    n = at_ref.shape[0]
    half = n // 2
    k0 = bounds_ref[i] // _CH
    k1 = (bounds_ref[i + 1] + _CH - 1) // _CH

    row_iota = jax.lax.broadcasted_iota(jnp.int32, (_T, _CH), 0)
    col_iota = jax.lax.broadcasted_iota(jnp.int32, (n, _CH), 0)

    junk = jnp.concatenate(
        [pltpu.matmul_pop(0, (half, _T), jnp.float32, m) for m in (0, 1)],
        axis=0)
    at_ref[...] = junk.astype(at_ref.dtype)

    def do_chunk(k):
        rv = rl_ref[pl.ds(k, 1), :] - i * _T
        cv = cl_ref[pl.ds(k, 1), :]
        d_oh = (row_iota == rv).astype(jnp.float8_e4m3fn)
        s_oh = (col_iota == cv).astype(jnp.float8_e4m3fn)
        for kt in range(_CH // 256):
            d_t = d_oh[:, kt * 256:(kt + 1) * 256]
            s_t = s_oh[:, kt * 256:(kt + 1) * 256]
            for m in (0, 1):
                pltpu.matmul_push_rhs(d_t, staging_register=0, mxu_index=m,
                                      transpose=True)
                pltpu.matmul_acc_lhs(0, s_t[m * half:(m + 1) * half, :],
                                     mxu_index=m, load_staged_rhs=0)

    def body(p_idx, _):
        do_chunk(2 * p_idx)
        do_chunk(2 * p_idx + 1)
        return _

    jax.lax.fori_loop(k0 // 2, (k1 + 1) // 2, body, 0)
    acc = jnp.concatenate(
        [pltpu.matmul_pop(0, (half, _T), jnp.float32, m) for m in (0, 1)],
        axis=0)
    at_ref[...] = acc.astype(at_ref.dtype)
    deg_ref[...] = jnp.sum(acc, axis=0, keepdims=True)


def _xw_kernel(x_ref, w_ref, dis_ref, o_ref):
    i = pl.program_id(0)
    x = x_ref[...].astype(jnp.bfloat16)
    w = w_ref[...].astype(jnp.bfloat16)
    dis_i = dis_ref[pl.ds(i * _T, _T), :]
    o_ref[...] = (dis_i * jnp.dot(
        x, w, preferred_element_type=jnp.float32
    )).astype(jnp.bfloat16)


def _h_kernel(at_ref, p_ref, w2_ref, b1_ref, dis_ref, q_ref):
    i = pl.program_id(0)
    h = jax.lax.dot_general(
        at_ref[...].astype(jnp.bfloat16), p_ref[...],
        (((0,), (0,)), ((), ())),
        preferred_element_type=jnp.float32)
    p_i = p_ref[pl.ds(i * _TB, _TB), :].astype(jnp.float32)
    dis_i = dis_ref[pl.ds(i * _TB, _TB), :]
    h = jnp.maximum(dis_i * (h + p_i) + b1_ref[...], 0.0)
    q = jnp.dot(h.astype(jnp.bfloat16), w2_ref[...],
                preferred_element_type=jnp.float32)
    q_ref[...] = (dis_i * q).astype(jnp.bfloat16)


def _out_kernel(at_ref, q_ref, b2_ref, dis_ref, o_ref):
    i = pl.program_id(0)
    o = jax.lax.dot_general(
        at_ref[...].astype(jnp.bfloat16), q_ref[...],
        (((0,), (0,)), ((), ())),
        preferred_element_type=jnp.float32)
    q_i = q_ref[pl.ds(i * _TB, _TB), :].astype(jnp.float32)
    dis_i = dis_ref[pl.ds(i * _TB, _TB), :]
    c = o_ref.shape[1]
    o_ref[...] = (dis_i * (o + q_i) + b2_ref[...])[:, :c]


def kernel(x, edge_index, w1, b1, w2, b2):
    n, f_in = x.shape
    hidden = w1.shape[1]
    c = w2.shape[1]
    c_pad = max(128, ((c + 127) // 128) * 128)
    e = edge_index.shape[1]
    n_tiles = n // _T
    e_pad = ((e + _CH - 1) // _CH) * _CH

    src = edge_index[0]
    dst = edge_index[1]
    lin = jnp.sort(jnp.concatenate(
        [dst * n + src, jnp.full((e_pad - e,), n * n, jnp.int32)]))
    tile_of_edge = dst // _T
    bounds = jnp.concatenate([
        jnp.zeros((1,), jnp.int32),
        jnp.cumsum(jnp.sum(
            tile_of_edge[None, :] == jnp.arange(n_tiles)[:, None],
            axis=1, dtype=jnp.int32))])
    rl = (lin // n).reshape(e_pad // _CH, _CH)
    cl = (lin % n).reshape(e_pad // _CH, _CH)

    w2p = jnp.zeros((hidden, c_pad), jnp.bfloat16).at[:, :c].set(
        w2.astype(jnp.bfloat16))
    b2p = jnp.zeros((1, c_pad), jnp.float32).at[:, :c].set(b2)

    grid = (n_tiles,)
    par = pltpu.CompilerParams(dimension_semantics=("parallel",))

    a_t, deg = pl.pallas_call(
        _build_a_kernel,
        out_shape=(jax.ShapeDtypeStruct((n, n), jnp.float8_e4m3fn),
                   jax.ShapeDtypeStruct((1, n), jnp.float32)),
        grid=grid,
        in_specs=[
            pl.BlockSpec(memory_space=pltpu.SMEM),
            pl.BlockSpec((e_pad // _CH, _CH), lambda i: (0, 0)),
            pl.BlockSpec((e_pad // _CH, _CH), lambda i: (0, 0)),
        ],
        out_specs=(pl.BlockSpec((n, _T), lambda i: (0, i)),
                   pl.BlockSpec((1, _T), lambda i: (0, i))),
        compiler_params=par,
    )(bounds, rl, cl)

    dis = (1.0 / jnp.sqrt(deg + 1.0)).reshape(n, 1)

    p_s = pl.pallas_call(
        _xw_kernel,
        out_shape=jax.ShapeDtypeStruct((n, hidden), jnp.bfloat16),
        grid=grid,
        in_specs=[
            pl.BlockSpec((_T, f_in), lambda i: (i, 0)),
            pl.BlockSpec((f_in, hidden), lambda i: (0, 0)),
            pl.BlockSpec((n, 1), lambda i: (0, 0)),
        ],
        out_specs=pl.BlockSpec((_T, hidden), lambda i: (i, 0)),
        compiler_params=par,
    )(x, w1, dis)

    grid_b = (n // _TB,)
    q = pl.pallas_call(
        _h_kernel,
        out_shape=jax.ShapeDtypeStruct((n, c_pad), jnp.bfloat16),
        grid=grid_b,
        in_specs=[
            pl.BlockSpec((n, _TB), lambda i: (0, i)),
            pl.BlockSpec((n, hidden), lambda i: (0, 0)),
            pl.BlockSpec((hidden, c_pad), lambda i: (0, 0)),
            pl.BlockSpec((1, hidden), lambda i: (0, 0)),
            pl.BlockSpec((n, 1), lambda i: (0, 0)),
        ],
        out_specs=pl.BlockSpec((_TB, c_pad), lambda i: (i, 0)),
        compiler_params=par,
    )(a_t, p_s, w2p, b1, dis)

    out = pl.pallas_call(
        _out_kernel,
        out_shape=jax.ShapeDtypeStruct((n, c), jnp.float32),
        grid=grid_b,
        in_specs=[
            pl.BlockSpec((n, _TB), lambda i: (0, i)),
            pl.BlockSpec((n, c_pad), lambda i: (0, 0)),
            pl.BlockSpec((1, c_pad), lambda i: (0, 0)),
            pl.BlockSpec((n, 1), lambda i: (0, 0)),
        ],
        out_specs=pl.BlockSpec((_TB, c), lambda i: (i, 0)),
        compiler_params=par,
    )(a_t, q, b2p, dis)

    return out
